# baseline (speedup 1.0000x reference)
import functools

import jax
import jax.numpy as jnp
from jax import lax
from jax.experimental import pallas as pl
from jax.experimental.pallas import tpu as pltpu
from jax.experimental.pallas import tpu_sc as plsc

_LANES = 16
_OUT_CHUNK = 4096
_UNROLL = 8


def _sc_geometry():
    info = plsc.get_sparse_core_info()
    return info.num_cores, info.num_subcores


@functools.partial(jax.jit, static_argnames=("num_cores", "num_subcores"))
def _lookup(indices, table, num_cores, num_subcores):
    num_workers = num_cores * num_subcores
    batch = indices.shape[0]
    table_t = table.T
    width, relations = table_t.shape
    rows_per_w = width // num_workers

    mesh = plsc.VectorSubcoreMesh(core_axis_name="c", subcore_axis_name="s")

    @functools.partial(
        pl.kernel,
        out_type=jax.ShapeDtypeStruct((width, batch), table.dtype),
        mesh=mesh,
        scratch_types=[
            pltpu.VMEM((relations,), table.dtype),
            pltpu.VMEM((batch,), jnp.int32),
            pltpu.VMEM((_OUT_CHUNK,), table.dtype),
            pltpu.VMEM((_OUT_CHUNK,), table.dtype),
            pltpu.SemaphoreType.DMA,
            pltpu.SemaphoreType.DMA,
            pltpu.SemaphoreType.DMA,
            pltpu.SemaphoreType.DMA,
        ],
        compiler_params=pltpu.CompilerParams(needs_layout_passes=False),
    )
    def gather_kernel(
        idx_hbm, table_hbm, out_hbm,
        row_v, idx_v, out_v0, out_v1,
        sem_row, sem_idx, sem_o0, sem_o1,
    ):
        wid = lax.axis_index("s") * num_cores + lax.axis_index("c")
        group = _UNROLL * _LANES
        n_chunks = batch // _OUT_CHUNK
        outs = [out_v0, out_v1]
        sems = [sem_o0, sem_o1]
        pending = [None, None]
        row_copy = None

        for p in range(rows_per_w):
            d = wid * rows_per_w + p
            if p == 0:
                idx_copy = pltpu.async_copy(idx_hbm, idx_v, sem_idx)

                @pl.when(wid % 2 == 1)
                def _stagger():
                    pl.delay(3500)

                row_copy = pltpu.async_copy(table_hbm.at[d], row_v, sem_row)
                idx_copy.wait()
            row_copy.wait()
            for h in range(n_chunks):
                b = h % 2
                if pending[b] is not None:
                    pending[b].wait()

                @plsc.parallel_loop(0, _OUT_CHUNK, step=_LANES, unroll=_UNROLL)
                def body(off, _h=h, _b=b):
                    iv = idx_v[pl.ds(_h * _OUT_CHUNK + off, _LANES)]
                    outs[_b][pl.ds(off, _LANES)] = plsc.load_gather(
                        row_v, [iv]
                    )
                if p + 1 == rows_per_w and h + 2 >= n_chunks:
                    pltpu.sync_copy(
                        outs[b],
                        out_hbm.at[d, pl.ds(h * _OUT_CHUNK, _OUT_CHUNK)],
                    )
                    pending[b] = None
                else:
                    pending[b] = pltpu.async_copy(
                        outs[b],
                        out_hbm.at[d, pl.ds(h * _OUT_CHUNK, _OUT_CHUNK)],
                        sems[b],
                    )
            if p + 1 < rows_per_w:
                row_copy = pltpu.async_copy(
                    table_hbm.at[d + 1], row_v, sem_row
                )

    out_t = gather_kernel(indices, table_t)
    return out_t.T


def kernel(indices, W_relation):
    num_cores, num_subcores = _sc_geometry()
    return _lookup(
        indices.astype(jnp.int32), W_relation, num_cores, num_subcores
    )


# speedup vs baseline: 1.1201x; 1.1201x over previous
import functools

import jax
import jax.numpy as jnp
from jax import lax
from jax.experimental import pallas as pl
from jax.experimental.pallas import tpu as pltpu
from jax.experimental.pallas import tpu_sc as plsc

_LANES = 16
_OUT_CHUNK = 4096
_UNROLL = 8


def _sc_geometry():
    info = plsc.get_sparse_core_info()
    return info.num_cores, info.num_subcores


@functools.partial(jax.jit, static_argnames=("num_cores", "num_subcores"))
def _lookup(indices, table, num_cores, num_subcores):
    num_workers = num_cores * num_subcores
    batch = indices.shape[0]
    table_t = table.T
    width, relations = table_t.shape
    rows_per_w = width // num_workers

    mesh = plsc.VectorSubcoreMesh(core_axis_name="c", subcore_axis_name="s")

    @functools.partial(
        pl.kernel,
        out_type=jax.ShapeDtypeStruct((width, batch), table.dtype),
        mesh=mesh,
        scratch_types=[
            pltpu.VMEM((relations,), table.dtype),
            pltpu.VMEM((batch,), jnp.int32),
            pltpu.VMEM((_OUT_CHUNK,), table.dtype),
            pltpu.VMEM((_OUT_CHUNK,), table.dtype),
            pltpu.VMEM_SHARED((batch,), jnp.int32),
            pltpu.SemaphoreType.DMA,
            pltpu.SemaphoreType.DMA,
            pltpu.SemaphoreType.DMA,
            pltpu.SemaphoreType.DMA,
        ],
        compiler_params=pltpu.CompilerParams(needs_layout_passes=False),
    )
    def gather_kernel(
        idx_hbm, table_hbm, out_hbm,
        row_v, idx_v, out_v0, out_v1, idx_sh,
        sem_row, sem_idx, sem_o0, sem_o1,
    ):
        wid = lax.axis_index("s") * num_cores + lax.axis_index("c")
        s_idx = lax.axis_index("s")
        group = _UNROLL * _LANES
        n_chunks = batch // _OUT_CHUNK
        outs = [out_v0, out_v1]
        sems = [sem_o0, sem_o1]
        pending = [None, None]
        row_copy = None

        for p in range(rows_per_w):
            d = wid * rows_per_w + p
            if p == 0:
                row_copy = pltpu.async_copy(table_hbm.at[d], row_v, sem_row)

                @pl.when(s_idx == 0)
                def _fetch_idx():
                    pltpu.async_copy(idx_hbm, idx_sh, sem_idx).wait()

                plsc.subcore_barrier()
                pltpu.sync_copy(idx_sh, idx_v)
            row_copy.wait()
            for h in range(n_chunks):
                b = h % 2
                if pending[b] is not None:
                    pending[b].wait()

                @plsc.parallel_loop(0, _OUT_CHUNK, step=_LANES, unroll=_UNROLL)
                def body(off, _h=h, _b=b):
                    iv = idx_v[pl.ds(_h * _OUT_CHUNK + off, _LANES)]
                    outs[_b][pl.ds(off, _LANES)] = plsc.load_gather(
                        row_v, [iv]
                    )
                if p + 1 == rows_per_w and h + 2 >= n_chunks:
                    pltpu.sync_copy(
                        outs[b],
                        out_hbm.at[d, pl.ds(h * _OUT_CHUNK, _OUT_CHUNK)],
                    )
                    pending[b] = None
                else:
                    pending[b] = pltpu.async_copy(
                        outs[b],
                        out_hbm.at[d, pl.ds(h * _OUT_CHUNK, _OUT_CHUNK)],
                        sems[b],
                    )
            if p + 1 < rows_per_w:
                row_copy = pltpu.async_copy(
                    table_hbm.at[d + 1], row_v, sem_row
                )

    out_t = gather_kernel(indices, table_t)
    return out_t.T


def kernel(indices, W_relation):
    num_cores, num_subcores = _sc_geometry()
    return _lookup(
        indices.astype(jnp.int32), W_relation, num_cores, num_subcores
    )
